# baseline (speedup 1.0000x reference)
import functools

import jax
import jax.numpy as jnp
from jax import lax
from jax.experimental import pallas as pl
from jax.experimental.pallas import tpu as pltpu
from jax.experimental.pallas import tpu_sc as plsc

_B, _S, _V = 1024, 50, 1000
_SP = 64
_NC, _NS = 2, 16
_NW = _NC * _NS
_BPW = _B // _NW
_G = (_S + 15) // 16
_CHI = _V - 16
_CLO = _V - 24


def _onehot_body(x_hbm, out_hbm, bufa, bufb, xv, sema, semb):
    wid = lax.axis_index("s") * _NC + lax.axis_index("c")
    pltpu.sync_copy(x_hbm.at[pl.ds(wid * _BPW * _SP, _BPW * _SP)], xv)

    zero16 = jnp.zeros((16,), jnp.float32)
    iota16 = lax.iota(jnp.int32, 16)
    bufs = (bufa, bufb)
    sems = (sema, semb)
    b0 = wid * _BPW

    def zinit(r, c):
        for q in range(2):
            for k in range(_V // 16):
                bufa[r * 2 + q, pl.ds(k * 16, 16)] = zero16
                bufb[r * 2 + q, pl.ds(k * 16, 16)] = zero16
            bufa[r * 2 + q, pl.ds(_V - 16, 16)] = zero16
            bufb[r * 2 + q, pl.ds(_V - 16, 16)] = zero16
        return c

    lax.fori_loop(0, _S // 2, zinit, 0)

    def put(e, buf):
        for g in range(_G):
            xv16 = xv[pl.ds(e * _SP + g * 16, 16)]
            cc16 = jnp.minimum(xv16 & jnp.int32(~15), _CLO)
            for l in range(min(16, _S - g * 16)):
                xs = xv16[l]
                row = g * 16 + l
                sel_hi = jnp.where(xs >= _CHI + 8, xs - _CHI, -1)
                v_hi = jnp.where(iota16 == sel_hi, 1.0, 0.0)
                buf[row, pl.ds(_CHI, 16)] = v_hi.astype(jnp.float32)
                cc = pl.multiple_of(cc16[l], 16)
                sel = jnp.where(xs != 0, xs - cc, -1)
                v = jnp.where(iota16 == sel, 1.0, 0.0)
                buf[row, pl.ds(cc, 16)] = v.astype(jnp.float32)

    def clear(e, buf):
        for g in range(_G):
            xv16 = xv[pl.ds(e * _SP + g * 16, 16)]
            cc16 = jnp.minimum(xv16 & jnp.int32(~15), _CLO)
            for l in range(min(16, _S - g * 16)):
                row = g * 16 + l
                buf[row, pl.ds(_CHI, 16)] = zero16
                cc = pl.multiple_of(cc16[l], 16)
                buf[row, pl.ds(cc, 16)] = zero16

    def fire(e, buf, sem):
        return pltpu.async_copy(buf, out_hbm.at[b0 + e], sem)

    for p in range(2):
        put(p, bufs[p])
        fire(p, bufs[p], sems[p])

    def ring(o, c):
        for p in range(2):
            e = o * 2 + p
            buf, sem = bufs[p], sems[p]
            pltpu.make_async_copy(buf, out_hbm.at[b0 + e - 2], sem).wait()
            clear(e - 2, buf)
            put(e, buf)
            fire(e, buf, sem)
        return c

    lax.fori_loop(1, _BPW // 2, ring, 0)

    for p in range(2):
        pltpu.make_async_copy(
            bufs[p], out_hbm.at[b0 + _BPW - 2 + p], sems[p]).wait()


_onehot_sc = functools.partial(
    pl.kernel,
    mesh=plsc.VectorSubcoreMesh(core_axis_name="c", subcore_axis_name="s"),
    out_type=jax.ShapeDtypeStruct((_B, _S, _V), jnp.float32),
    compiler_params=pltpu.CompilerParams(use_tc_tiling_on_sc=True),
    scratch_types=[
        pltpu.VMEM((_S, _V), jnp.float32),
        pltpu.VMEM((_S, _V), jnp.float32),
        pltpu.VMEM((_BPW * _SP,), jnp.int32),
        pltpu.SemaphoreType.DMA,
        pltpu.SemaphoreType.DMA,
    ],
)(_onehot_body)


@jax.jit
def kernel(x, W):
    del W
    xp = jnp.pad(x.astype(jnp.int32), ((0, 0), (0, _SP - _S)))
    return _onehot_sc(xp.reshape(_B * _SP))


# speedup vs baseline: 3.1842x; 3.1842x over previous
import functools

import jax
import jax.numpy as jnp
from jax import lax
from jax.experimental import pallas as pl
from jax.experimental.pallas import tpu as pltpu
from jax.experimental.pallas import tpu_sc as plsc

_B, _S, _V = 1024, 50, 1000
_NC, _NS = 2, 16
_NW = _NC * _NS
_BT = 128
_NU = _S * (_B // _BT)
_UPW = _NU // _NW
_NSUB = 5
_KSUB = _V // _NSUB


def _xoff(u):
    return pl.multiple_of((u >> 3) * _B + (u & 7) * _BT, _BT)


def _onehot_body(xt_hbm, out_hbm, buf, xbuf, rowbuf, xsem, dsem):
    wid = lax.axis_index("s") * _NC + lax.axis_index("c")
    u0 = wid * _UPW + jnp.minimum(wid, _NU - _NW * _UPW)
    n = _UPW + jnp.where(wid < _NU - _NW * _UPW, 1, 0)

    zero16 = jnp.zeros((16,), jnp.float32)
    iota16 = lax.iota(jnp.int32, 16)

    def zinit(r, c):
        for k in range(_BT // 16):
            buf[r, pl.ds(k * 16, 16)] = zero16
        return c

    lax.fori_loop(0, _V, zinit, 0)

    pltpu.async_copy(xt_hbm.at[pl.ds(_xoff(u0), _BT)], xbuf, xsem).wait()

    def unit(u, c):
        s = u >> 3
        beta = u & 7
        xvs = [xbuf[pl.ds(g * 16, 16)] for g in range(_BT // 16)]
        nxt = jnp.minimum(u + 1, _NU - 1)
        nxt_cp = pltpu.async_copy(
            xt_hbm.at[pl.ds(_xoff(nxt), _BT)], xbuf, xsem)

        for g in range(_BT // 16):
            xv16 = xvs[g]
            in16 = xv16 != 0
            rows16 = jnp.where(in16, xv16, 0)
            sel16 = jnp.where(in16, iota16, -1)
            rowbuf[pl.ds(g * 16, 16)] = rows16
            for l in range(16):
                rs = rows16[l]
                pat = jnp.where(iota16 == sel16[l], 1.0, 0.0)
                w = buf[rs, pl.ds(g * 16, 16)]
                buf[rs, pl.ds(g * 16, 16)] = jnp.maximum(
                    w, pat.astype(jnp.float32))

        subs = []
        for j in range(_NSUB):
            subs.append(pltpu.async_copy(
                buf.at[pl.ds(j * _KSUB, _KSUB)],
                out_hbm.at[s, pl.ds(j * _KSUB, _KSUB),
                           pl.ds(pl.multiple_of(beta * _BT, _BT), _BT)],
                dsem))
        for cp in subs:
            cp.wait()

        for g in range(_BT // 16):
            rows16 = rowbuf[pl.ds(g * 16, 16)]
            for l in range(16):
                buf[rows16[l], pl.ds(g * 16, 16)] = zero16

        nxt_cp.wait()
        return c

    lax.fori_loop(u0, u0 + n, unit, 0)


_onehot_sc = functools.partial(
    pl.kernel,
    mesh=plsc.VectorSubcoreMesh(core_axis_name="c", subcore_axis_name="s"),
    out_type=jax.ShapeDtypeStruct((_S, _V, _B), jnp.float32),
    scratch_types=[
        pltpu.VMEM((_V, _BT), jnp.float32),
        pltpu.VMEM((_BT,), jnp.int32),
        pltpu.VMEM((_BT,), jnp.int32),
        pltpu.SemaphoreType.DMA,
        pltpu.SemaphoreType.DMA,
    ],
)(_onehot_body)


@jax.jit
def kernel(x, W):
    del W
    xt = jnp.transpose(x.astype(jnp.int32)).reshape(_S * _B)
    out_t = _onehot_sc(xt)
    return jnp.transpose(out_t, (2, 0, 1))
